# baseline (speedup 1.0000x reference)
import functools

import jax
import jax.numpy as jnp
from jax import lax
from jax.experimental import pallas as pl
from jax.experimental.pallas import tpu as pltpu
from jax.experimental.pallas import tpu_sc as plsc

EMBED_DIM = 64
OUT_DIM = 128

_NC, _NS = 2, 16
_NW = _NC * _NS
_CHUNK = 512


_FLAT_TR = 128


def _sc_flatten(times2d, n_rows):
    bsz, seq = times2d.shape
    tr_per_w = bsz // _NW
    n_steps = tr_per_w // _FLAT_TR
    offs = list(range(0, seq - 15, 16))
    if offs[-1] + 16 < seq:
        offs.append(seq - 16)
    mesh = plsc.VectorSubcoreMesh(core_axis_name="c", subcore_axis_name="s")

    @functools.partial(
        pl.kernel,
        mesh=mesh,
        out_type=jax.ShapeDtypeStruct((n_rows // 128, 128), jnp.int32),
        scratch_types=[
            pltpu.VMEM((_FLAT_TR, seq), jnp.int32),
            pltpu.VMEM((_FLAT_TR * seq // 128, 128), jnp.int32),
        ],
        compiler_params=pltpu.CompilerParams(needs_layout_passes=False),
    )
    def flat_kernel(times_hbm, out_hbm, in_v, flat_v):
        wid = lax.axis_index("s") * _NC + lax.axis_index("c")
        r0 = pl.multiple_of(wid * tr_per_w, 8)
        f0 = pl.multiple_of(wid * (tr_per_w * seq // 128), 8)
        iota16 = lax.iota(jnp.int32, 16)
        pltpu.sync_copy(times_hbm.at[pl.ds(r0, _FLAT_TR)], in_v)

        def body(k, carry):
            for o in offs:
                p = k * seq + o + iota16
                plsc.store_scatter(
                    flat_v,
                    [p >> 7, p & 127],
                    in_v[k, pl.ds(o, 16)] * 2,
                )
            return carry

        lax.fori_loop(0, _FLAT_TR, body, 0)
        pltpu.sync_copy(
            flat_v, out_hbm.at[pl.ds(f0, _FLAT_TR * seq // 128)]
        )

    return flat_kernel(times2d)


def _sc_gather(table, flat_idx, n_rows, kfr):
    rows_per_w = n_rows // _NW
    n_chunks = rows_per_w // (kfr * 128)
    mesh = plsc.VectorSubcoreMesh(core_axis_name="c", subcore_axis_name="s")

    @functools.partial(
        pl.kernel,
        mesh=mesh,
        out_type=jax.ShapeDtypeStruct(
            (n_rows // 128, 128, 2 * EMBED_DIM), jnp.float32
        ),
        scratch_types=[
            pltpu.VMEM((kfr, 128), jnp.int32),
            pltpu.VMEM((kfr, 128, EMBED_DIM), jnp.float32),
            pltpu.SemaphoreType.DMA,
        ],
        compiler_params=pltpu.CompilerParams(use_tc_tiling_on_sc=False),
    )
    def gather_kernel(table_hbm, idx_hbm, out_hbm, idx_v, rows_v, sem):
        wid = lax.axis_index("s") * _NC + lax.axis_index("c")
        base = wid * (rows_per_w // 128)

        def body(i, carry):
            fr0 = base + i * kfr
            pltpu.sync_copy(idx_hbm.at[pl.ds(fr0, kfr)], idx_v)
            copies = [
                pltpu.async_copy(
                    table_hbm.at[idx_v.at[k]],
                    rows_v.at[k],
                    sem,
                )
                for k in range(kfr)
            ]
            for c in copies:
                c.wait()
            pltpu.sync_copy(
                rows_v,
                out_hbm.at[pl.ds(fr0, kfr), :, pl.ds(0, EMBED_DIM)],
            )
            return carry

        lax.fori_loop(0, n_chunks, body, 0)

    return gather_kernel(table, flat_idx)


_SQRT_HALF = 0.7071067811865476


def _proj_body(e_ref, w_ref, b_ref, o_ref):
    x = e_ref[...][:, :EMBED_DIM]
    h = 0.5 * x * (1.0 + lax.erf(x * _SQRT_HALF))
    acc = lax.dot_general(
        h, w_ref[...], (((1,), (1,)), ((), ())),
        preferred_element_type=jnp.float32,
    )
    o_ref[...] = acc + b_ref[...]


def _proj_body_acc(big_ref, e_ref, w_ref, b_ref, o_ref):
    _proj_body(e_ref, w_ref, b_ref, o_ref)


def _tc_project_chunk(big, e, w, b2d, n_rows, blk, c, nch):
    bpc = n_rows // nch // blk
    grid = (bpc,)
    e_spec = pl.BlockSpec((blk, 2 * EMBED_DIM), lambda i: (i, 0))
    w_spec = pl.BlockSpec((OUT_DIM, EMBED_DIM), lambda i: (0, 0))
    b_spec = pl.BlockSpec((1, OUT_DIM), lambda i: (0, 0))
    out_spec = pl.BlockSpec((blk, OUT_DIM), lambda i: (i + c * bpc, 0))
    out_shape = jax.ShapeDtypeStruct((n_rows, OUT_DIM), jnp.float32)
    if big is None:
        return pl.pallas_call(
            _proj_body,
            grid=grid,
            in_specs=[e_spec, w_spec, b_spec],
            out_specs=out_spec,
            out_shape=out_shape,
        )(e, w, b2d)
    return pl.pallas_call(
        _proj_body_acc,
        grid=grid,
        in_specs=[
            pl.BlockSpec(memory_space=pl.ANY),
            e_spec,
            w_spec,
            b_spec,
        ],
        out_specs=out_spec,
        out_shape=out_shape,
        input_output_aliases={0: 0},
    )(big, e, w, b2d)


def _padT_body(t_ref, i_ref, o_ref):
    xt = lax.dot_general(
        t_ref[...], i_ref[...], (((0,), (0,)), ((), ())),
        preferred_element_type=jnp.float32,
    )
    o_ref[...] = jnp.concatenate([xt, xt], axis=1)


def _tc_padT(tableT, nblk, tblk):
    eye = jnp.eye(EMBED_DIM, dtype=jnp.float32)
    return pl.pallas_call(
        _padT_body,
        grid=(nblk,),
        in_specs=[
            pl.BlockSpec((EMBED_DIM, tblk), lambda i: (0, i)),
            pl.BlockSpec((EMBED_DIM, EMBED_DIM), lambda i: (0, 0)),
        ],
        out_specs=pl.BlockSpec((tblk, 2 * EMBED_DIM), lambda i: (i, 0)),
        out_shape=jax.ShapeDtypeStruct((nblk * tblk, 2 * EMBED_DIM), jnp.float32),
    )(tableT, eye)


_NCH = 4


def kernel(times, table, W, b):
    bsz, seq = times.shape
    n_rows = bsz * seq
    flat_idx = _sc_flatten(times.astype(jnp.int32), n_rows)
    nv = table.shape[0]
    tblk = 2048
    nblk = (nv + tblk - 1) // tblk
    tpad = _tc_padT(table.T, nblk, tblk)
    table2 = tpad.reshape(2 * nblk * tblk, EMBED_DIM)
    frc = n_rows // 128 // _NCH
    rows_c = n_rows // _NCH
    kfr = 4 if (rows_c // _NW) % 512 == 0 else 2
    b2d = b.reshape(1, OUT_DIM)
    big = None
    for c in range(_NCH):
        fc = lax.slice(flat_idx, (c * frc, 0), ((c + 1) * frc, 128))
        e3 = _sc_gather(table2, fc, rows_c, kfr)
        e = e3.reshape(rows_c, 2 * EMBED_DIM)
        big = _tc_project_chunk(big, e, W, b2d, n_rows, 4096, c, _NCH)
    return big.reshape(bsz, seq, OUT_DIM)


# speedup vs baseline: 1.0373x; 1.0373x over previous
import functools

import jax
import jax.numpy as jnp
from jax import lax
from jax.experimental import pallas as pl
from jax.experimental.pallas import tpu as pltpu
from jax.experimental.pallas import tpu_sc as plsc

EMBED_DIM = 64
OUT_DIM = 128

_NC, _NS = 2, 16
_NW = _NC * _NS
_CHUNK = 512


_FLAT_TR = 128


def _sc_flatten(times2d, n_rows):
    bsz, seq = times2d.shape
    tr_per_w = bsz // _NW
    n_steps = tr_per_w // _FLAT_TR
    offs = list(range(0, seq - 15, 16))
    if offs[-1] + 16 < seq:
        offs.append(seq - 16)
    mesh = plsc.VectorSubcoreMesh(core_axis_name="c", subcore_axis_name="s")

    @functools.partial(
        pl.kernel,
        mesh=mesh,
        out_type=jax.ShapeDtypeStruct((n_rows // 128, 128), jnp.int32),
        scratch_types=[
            pltpu.VMEM((_FLAT_TR, seq), jnp.int32),
            pltpu.VMEM((_FLAT_TR * seq // 128, 128), jnp.int32),
        ],
        compiler_params=pltpu.CompilerParams(needs_layout_passes=False),
    )
    def flat_kernel(times_hbm, out_hbm, in_v, flat_v):
        wid = lax.axis_index("s") * _NC + lax.axis_index("c")
        r0 = pl.multiple_of(wid * tr_per_w, 8)
        f0 = pl.multiple_of(wid * (tr_per_w * seq // 128), 8)
        iota16 = lax.iota(jnp.int32, 16)
        pltpu.sync_copy(times_hbm.at[pl.ds(r0, _FLAT_TR)], in_v)

        def body(k, carry):
            for o in offs:
                p = k * seq + o + iota16
                plsc.store_scatter(
                    flat_v,
                    [p >> 7, p & 127],
                    in_v[k, pl.ds(o, 16)] * 2,
                )
            return carry

        lax.fori_loop(0, _FLAT_TR, body, 0)
        pltpu.sync_copy(
            flat_v, out_hbm.at[pl.ds(f0, _FLAT_TR * seq // 128)]
        )

    return flat_kernel(times2d)


def _sc_gather(table, flat_idx, n_rows, kfr):
    rows_per_w = n_rows // _NW
    n_chunks = rows_per_w // (kfr * 128)
    mesh = plsc.VectorSubcoreMesh(core_axis_name="c", subcore_axis_name="s")

    @functools.partial(
        pl.kernel,
        mesh=mesh,
        out_type=jax.ShapeDtypeStruct(
            (n_rows // 128, 128, 2 * EMBED_DIM), jnp.float32
        ),
        scratch_types=[
            pltpu.VMEM((kfr, 128), jnp.int32),
            pltpu.VMEM((kfr, 128, EMBED_DIM), jnp.float32),
            pltpu.SemaphoreType.DMA,
        ],
        compiler_params=pltpu.CompilerParams(use_tc_tiling_on_sc=False),
    )
    def gather_kernel(table_hbm, idx_hbm, out_hbm, idx_v, rows_v, sem):
        wid = lax.axis_index("s") * _NC + lax.axis_index("c")
        base = wid * (rows_per_w // 128)

        def body(i, carry):
            fr0 = base + i * kfr
            pltpu.sync_copy(idx_hbm.at[pl.ds(fr0, kfr)], idx_v)
            copies = [
                pltpu.async_copy(
                    table_hbm.at[idx_v.at[k]],
                    rows_v.at[k],
                    sem,
                )
                for k in range(kfr)
            ]
            for c in copies:
                c.wait()
            pltpu.sync_copy(
                rows_v,
                out_hbm.at[pl.ds(fr0, kfr), :, pl.ds(0, EMBED_DIM)],
            )
            return carry

        lax.fori_loop(0, n_chunks, body, 0)

    return gather_kernel(table, flat_idx)


_SQRT_HALF = 0.7071067811865476


def _proj_body(e_ref, w_ref, b_ref, o_ref):
    x = e_ref[...][:, :EMBED_DIM]
    h = 0.5 * x * (1.0 + lax.erf(x * _SQRT_HALF))
    acc = lax.dot_general(
        h, w_ref[...], (((1,), (1,)), ((), ())),
        preferred_element_type=jnp.float32,
    )
    o_ref[...] = acc + b_ref[...]


def _proj_body_acc(big_ref, e_ref, w_ref, b_ref, o_ref):
    _proj_body(e_ref, w_ref, b_ref, o_ref)


def _tc_project_chunk(big, e, w, b2d, n_rows, blk, c, nch):
    bpc = n_rows // nch // blk
    grid = (bpc,)
    e_spec = pl.BlockSpec((blk, 2 * EMBED_DIM), lambda i: (i, 0))
    w_spec = pl.BlockSpec((OUT_DIM, EMBED_DIM), lambda i: (0, 0))
    b_spec = pl.BlockSpec((1, OUT_DIM), lambda i: (0, 0))
    out_spec = pl.BlockSpec((blk, OUT_DIM), lambda i: (i + c * bpc, 0))
    out_shape = jax.ShapeDtypeStruct((n_rows, OUT_DIM), jnp.float32)
    if big is None:
        return pl.pallas_call(
            _proj_body,
            grid=grid,
            in_specs=[e_spec, w_spec, b_spec],
            out_specs=out_spec,
            out_shape=out_shape,
        )(e, w, b2d)
    return pl.pallas_call(
        _proj_body_acc,
        grid=grid,
        in_specs=[
            pl.BlockSpec(memory_space=pl.ANY),
            e_spec,
            w_spec,
            b_spec,
        ],
        out_specs=out_spec,
        out_shape=out_shape,
        input_output_aliases={0: 0},
    )(big, e, w, b2d)


def _padT_body(t_ref, i_ref, o_ref):
    o_ref[...] = lax.dot_general(
        t_ref[...], i_ref[...], (((0,), (0,)), ((), ())),
        preferred_element_type=jnp.float32,
        precision=lax.Precision.HIGHEST,
    )


def _tc_padT(tableT, nblk, tblk):
    eye = jnp.eye(EMBED_DIM, 2 * EMBED_DIM, dtype=jnp.float32)
    return pl.pallas_call(
        _padT_body,
        grid=(nblk,),
        in_specs=[
            pl.BlockSpec((EMBED_DIM, tblk), lambda i: (0, i)),
            pl.BlockSpec((EMBED_DIM, 2 * EMBED_DIM), lambda i: (0, 0)),
        ],
        out_specs=pl.BlockSpec((tblk, 2 * EMBED_DIM), lambda i: (i, 0)),
        out_shape=jax.ShapeDtypeStruct((nblk * tblk, 2 * EMBED_DIM), jnp.float32),
    )(tableT, eye)


_NCH = 4


def kernel(times, table, W, b):
    bsz, seq = times.shape
    n_rows = bsz * seq
    flat_idx = _sc_flatten(times.astype(jnp.int32), n_rows)
    nv = table.shape[0]
    tblk = 4096
    nblk = (nv + tblk - 1) // tblk
    tpad = _tc_padT(table.T, nblk, tblk)
    table2 = tpad.reshape(2 * nblk * tblk, EMBED_DIM)
    frc = n_rows // 128 // _NCH
    rows_c = n_rows // _NCH
    kfr = 4 if (rows_c // _NW) % 512 == 0 else 2
    b2d = b.reshape(1, OUT_DIM)
    big = None
    for c in range(_NCH):
        fc = lax.slice(flat_idx, (c * frc, 0), ((c + 1) * frc, 128))
        e3 = _sc_gather(table2, fc, rows_c, kfr)
        e = e3.reshape(rows_c, 2 * EMBED_DIM)
        big = _tc_project_chunk(big, e, W, b2d, n_rows, 4096, c, _NCH)
    return big.reshape(bsz, seq, OUT_DIM)


# speedup vs baseline: 1.0896x; 1.0504x over previous
import functools

import jax
import jax.numpy as jnp
from jax import lax
from jax.experimental import pallas as pl
from jax.experimental.pallas import tpu as pltpu
from jax.experimental.pallas import tpu_sc as plsc

EMBED_DIM = 64
OUT_DIM = 128

_NC, _NS = 2, 16
_NW = _NC * _NS
_CHUNK = 512


_FLAT_TR = 128


def _sc_flatten(times2d, n_rows):
    bsz, seq = times2d.shape
    tr_per_w = bsz // _NW
    n_steps = tr_per_w // _FLAT_TR
    offs = list(range(0, seq - 15, 16))
    if offs[-1] + 16 < seq:
        offs.append(seq - 16)
    mesh = plsc.VectorSubcoreMesh(core_axis_name="c", subcore_axis_name="s")

    @functools.partial(
        pl.kernel,
        mesh=mesh,
        out_type=jax.ShapeDtypeStruct((n_rows // 128, 128), jnp.int32),
        scratch_types=[
            pltpu.VMEM((_FLAT_TR, seq), jnp.int32),
            pltpu.VMEM((_FLAT_TR * seq // 128, 128), jnp.int32),
        ],
        compiler_params=pltpu.CompilerParams(needs_layout_passes=False),
    )
    def flat_kernel(times_hbm, out_hbm, in_v, flat_v):
        wid = lax.axis_index("s") * _NC + lax.axis_index("c")
        r0 = pl.multiple_of(wid * tr_per_w, 8)
        f0 = pl.multiple_of(wid * (tr_per_w * seq // 128), 8)
        iota16 = lax.iota(jnp.int32, 16)
        pltpu.sync_copy(times_hbm.at[pl.ds(r0, _FLAT_TR)], in_v)

        def body(k, carry):
            for o in offs:
                p = k * seq + o + iota16
                plsc.store_scatter(
                    flat_v,
                    [p >> 7, p & 127],
                    in_v[k, pl.ds(o, 16)] * 2,
                )
            return carry

        lax.fori_loop(0, _FLAT_TR, body, 0)
        pltpu.sync_copy(
            flat_v, out_hbm.at[pl.ds(f0, _FLAT_TR * seq // 128)]
        )

    return flat_kernel(times2d)


def _sc_gather(table, flat_idx, n_rows, kfr):
    rows_per_w = n_rows // _NW
    n_chunks = rows_per_w // (kfr * 128)
    mesh = plsc.VectorSubcoreMesh(core_axis_name="c", subcore_axis_name="s")

    @functools.partial(
        pl.kernel,
        mesh=mesh,
        out_type=jax.ShapeDtypeStruct(
            (n_rows // 128, 128, 2 * EMBED_DIM), jnp.float32
        ),
        scratch_types=[
            pltpu.VMEM((kfr, 128), jnp.int32),
            pltpu.VMEM((kfr, 128, EMBED_DIM), jnp.float32),
            pltpu.SemaphoreType.DMA,
        ],
        compiler_params=pltpu.CompilerParams(use_tc_tiling_on_sc=False),
    )
    def gather_kernel(table_hbm, idx_hbm, out_hbm, idx_v, rows_v, sem):
        wid = lax.axis_index("s") * _NC + lax.axis_index("c")
        base = wid * (rows_per_w // 128)

        def body(i, carry):
            fr0 = base + i * kfr
            pltpu.sync_copy(idx_hbm.at[pl.ds(fr0, kfr)], idx_v)
            copies = [
                pltpu.async_copy(
                    table_hbm.at[idx_v.at[k]],
                    rows_v.at[k],
                    sem,
                )
                for k in range(kfr)
            ]
            for c in copies:
                c.wait()
            pltpu.sync_copy(
                rows_v,
                out_hbm.at[pl.ds(fr0, kfr), :, pl.ds(0, EMBED_DIM)],
            )
            return carry

        lax.fori_loop(0, n_chunks, body, 0)

    return gather_kernel(table, flat_idx)


_SQRT_HALF = 0.7071067811865476


def _proj_body(e_ref, w_ref, b_ref, o_ref):
    x = e_ref[...][:, :EMBED_DIM]
    h = 0.5 * x * (1.0 + lax.erf(x * _SQRT_HALF))
    acc = lax.dot_general(
        h, w_ref[...], (((1,), (1,)), ((), ())),
        preferred_element_type=jnp.float32,
    )
    o_ref[...] = acc + b_ref[...]


def _proj_body_acc(big_ref, e_ref, w_ref, b_ref, o_ref):
    _proj_body(e_ref, w_ref, b_ref, o_ref)


def _tc_project_chunk(big, e, w, b2d, n_rows, blk, c, nch):
    bpc = n_rows // nch // blk
    grid = (bpc,)
    e_spec = pl.BlockSpec((blk, 2 * EMBED_DIM), lambda i: (i, 0))
    w_spec = pl.BlockSpec((OUT_DIM, EMBED_DIM), lambda i: (0, 0))
    b_spec = pl.BlockSpec((1, OUT_DIM), lambda i: (0, 0))
    out_spec = pl.BlockSpec((blk, OUT_DIM), lambda i: (i + c * bpc, 0))
    out_shape = jax.ShapeDtypeStruct((n_rows, OUT_DIM), jnp.float32)
    if big is None:
        return pl.pallas_call(
            _proj_body,
            grid=grid,
            in_specs=[e_spec, w_spec, b_spec],
            out_specs=out_spec,
            out_shape=out_shape,
        )(e, w, b2d)
    return pl.pallas_call(
        _proj_body_acc,
        grid=grid,
        in_specs=[
            pl.BlockSpec(memory_space=pl.ANY),
            e_spec,
            w_spec,
            b_spec,
        ],
        out_specs=out_spec,
        out_shape=out_shape,
        input_output_aliases={0: 0},
    )(big, e, w, b2d)


def _padT_body(t_ref, i_ref, o_ref):
    o_ref[...] = lax.dot_general(
        t_ref[...], i_ref[...], (((0,), (0,)), ((), ())),
        preferred_element_type=jnp.float32,
        precision=lax.Precision.HIGHEST,
    )


def _tc_padT(tableT, nblk, tblk):
    eye = jnp.eye(EMBED_DIM, 2 * EMBED_DIM, dtype=jnp.float32)
    return pl.pallas_call(
        _padT_body,
        grid=(nblk,),
        in_specs=[
            pl.BlockSpec((EMBED_DIM, tblk), lambda i: (0, i)),
            pl.BlockSpec((EMBED_DIM, 2 * EMBED_DIM), lambda i: (0, 0)),
        ],
        out_specs=pl.BlockSpec((tblk, 2 * EMBED_DIM), lambda i: (i, 0)),
        out_shape=jax.ShapeDtypeStruct((nblk * tblk, 2 * EMBED_DIM), jnp.float32),
    )(tableT, eye)


_NCH = 4


def kernel(times, table, W, b):
    bsz, seq = times.shape
    n_rows = bsz * seq
    flat_idx = _sc_flatten(times.astype(jnp.int32), n_rows)
    nv = table.shape[0]
    tblk = 8192
    nblk = (nv + tblk - 1) // tblk
    tpad = _tc_padT(table.T, nblk, tblk)
    table2 = tpad.reshape(2 * nblk * tblk, EMBED_DIM)
    frc = n_rows // 128 // _NCH
    rows_c = n_rows // _NCH
    kfr = 4 if (rows_c // _NW) % 512 == 0 else 2
    b2d = b.reshape(1, OUT_DIM)
    big = None
    for c in range(_NCH):
        fc = lax.slice(flat_idx, (c * frc, 0), ((c + 1) * frc, 128))
        e3 = _sc_gather(table2, fc, rows_c, kfr)
        e = e3.reshape(rows_c, 2 * EMBED_DIM)
        big = _tc_project_chunk(big, e, W, b2d, n_rows, 8192, c, _NCH)
    return big.reshape(bsz, seq, OUT_DIM)
